# baseline (speedup 1.0000x reference)
import jax
import jax.numpy as jnp
from jax.experimental import pallas as pl

_H = 128
_W = 128
_C = 128
_NH = 4
_HD = 32
_K = 5
_S = 9
_L = _K * _K
_R = 4


def _proj_kernel(x_ref, w_ref, b_ref, q_ref, k_ref):
    xb = x_ref[...]
    r, w, c = xb.shape
    qk = jnp.dot(xb.reshape(r * w, c), w_ref[...],
                 preferred_element_type=jnp.float32) + b_ref[0][None, :]
    q = qk[:, :c].reshape(r, w, c).transpose(0, 2, 1)
    k = qk[:, c:].reshape(r, w, c).transpose(0, 2, 1)
    q_ref[...] = q
    k_ref[...] = k


def _roll_lanes(a, shift):
    shift %= a.shape[-1]
    if shift == 0:
        return a
    return jnp.concatenate([a[..., -shift:], a[..., :-shift]], axis=-1)


def _gather_cols(a, dx, jj):
    rolled = _roll_lanes(a, 2 - dx)
    lo = a[..., dx:dx + 1]
    hi = a[..., _W - _K + dx:_W - _K + dx + 1]
    return jnp.where(jj < 2, lo, jnp.where(jj > _W - 3, hi, rolled))


def _attn_kernel(q_ref, k_ref, sims_ref, sinds_ref, out_ref):
    i = pl.program_id(0)
    jj1 = jax.lax.broadcasted_iota(jnp.int32, (1, 1, _W), 2)
    for r in range(_R):
        ig = i * _R + r
        rstart = jnp.clip(ig - 2, 0, _H - _K)
        kwin = k_ref[pl.ds(rstart, _K)]
        swin = sims_ref[pl.ds(rstart, _K)]
        iwin = sinds_ref[pl.ds(rstart, _K)]
        irow = sinds_ref[pl.ds(ig, 1)][0]
        qrow = q_ref[r]

        kn_parts, sn_parts, in_parts = [], [], []
        for dx in range(_K):
            kn_parts.append(_gather_cols(kwin, dx, jj1))
            sn_parts.append(_gather_cols(swin, dx, jj1))
            in_parts.append(_gather_cols(iwin, dx, jj1))
        kn = jnp.stack(kn_parts, axis=1)
        sn = jnp.stack(sn_parts, axis=1).reshape(_L, _S, _W)
        idn = jnp.stack(in_parts, axis=1).reshape(_L, _S, _W)

        prod = kn * qrow[None, None]
        qkdot = prod.reshape(_K, _K, _NH, _HD, _W).sum(axis=3).reshape(_L, _NH, _W)

        eq = (idn[:, :, None, :] == irow[None, None, :, :])
        pk = jnp.sum(sn[:, :, None, :] * eq.astype(jnp.float32), axis=1)
        cond = pk > 0.0
        logpk = jnp.log(jnp.where(cond, pk, 1.0))

        att = jnp.where(cond[:, None], qkdot[:, :, None] + logpk[:, None],
                        -jnp.inf)
        out_ref[:, :, r] = att.transpose(1, 2, 3, 0)


def kernel(x, sims, sinds, qk_w, qk_b):
    x2 = x[0]
    sims_t = jnp.transpose(sims[0], (1, 0, 2))
    sinds_t = jnp.transpose(sinds[0], (1, 0, 2))
    b2 = qk_b.reshape(1, 2 * _C)

    nsteps = _H // _R
    q_t, k_t = pl.pallas_call(
        _proj_kernel,
        grid=(nsteps,),
        in_specs=[
            pl.BlockSpec((_R, _W, _C), lambda i: (i, 0, 0)),
            pl.BlockSpec((_C, 2 * _C), lambda i: (0, 0)),
            pl.BlockSpec((1, 2 * _C), lambda i: (0, 0)),
        ],
        out_specs=[
            pl.BlockSpec((_R, _C, _W), lambda i: (i, 0, 0)),
            pl.BlockSpec((_R, _C, _W), lambda i: (i, 0, 0)),
        ],
        out_shape=[
            jax.ShapeDtypeStruct((_H, _C, _W), jnp.float32),
            jax.ShapeDtypeStruct((_H, _C, _W), jnp.float32),
        ],
    )(x2, qk_w, b2)

    attn = pl.pallas_call(
        _attn_kernel,
        grid=(nsteps,),
        in_specs=[
            pl.BlockSpec((_R, _C, _W), lambda i: (i, 0, 0)),
            pl.BlockSpec((_H, _C, _W), lambda i: (0, 0, 0)),
            pl.BlockSpec((_H, _S, _W), lambda i: (0, 0, 0)),
            pl.BlockSpec((_H, _S, _W), lambda i: (0, 0, 0)),
        ],
        out_specs=pl.BlockSpec((_NH, _S, _R, _W, _L), lambda i: (0, 0, i, 0, 0)),
        out_shape=jax.ShapeDtypeStruct((_NH, _S, _H, _W, _L), jnp.float32),
    )(q_t, k_t, sims_t, sinds_t)

    return attn[None]


# speedup vs baseline: 2.8902x; 2.8902x over previous
import jax
import jax.numpy as jnp
from jax.experimental import pallas as pl

_H = 128
_W = 128
_C = 128
_NH = 4
_HD = 32
_K = 5
_S = 9
_L = _K * _K
_R = 4


def _proj_kernel(x_ref, w_ref, b_ref, q_ref, k_ref):
    xb = x_ref[...]
    r, w, c = xb.shape
    qk = jnp.dot(xb.reshape(r * w, c), w_ref[...],
                 preferred_element_type=jnp.float32) + b_ref[0][None, :]
    q = qk[:, :c].reshape(r, w, c).transpose(0, 2, 1)
    k = qk[:, c:].reshape(r, w, c).transpose(0, 2, 1)
    q_ref[...] = q
    k_ref[...] = k


def _roll_lanes(a, shift):
    shift %= a.shape[-1]
    if shift == 0:
        return a
    return jnp.concatenate([a[..., -shift:], a[..., :-shift]], axis=-1)


def _gather_cols(a, dx, jj):
    rolled = _roll_lanes(a, 2 - dx)
    lo = a[..., dx:dx + 1]
    hi = a[..., _W - _K + dx:_W - _K + dx + 1]
    return jnp.where(jj < 2, lo, jnp.where(jj > _W - 3, hi, rolled))


def _attn_rows(q_ref, k_ref, sims_ref, sinds_ref, out_ref, wstart, lrs, locs):
    jj1 = jax.lax.broadcasted_iota(jnp.int32, (1, 1, _W), 2)
    kwin8 = k_ref[pl.ds(wstart, _R + 4)]
    swin8 = sims_ref[pl.ds(wstart, _R + 4)]
    iwin8 = sinds_ref[pl.ds(wstart, _R + 4)]
    for r in range(_R):
        lr = lrs[r]
        kwin = kwin8[lr:lr + _K]
        swin = swin8[lr:lr + _K]
        iwin = iwin8[lr:lr + _K]
        irow = iwin8[locs[r]]
        qrow = q_ref[r]

        red_parts, sn_parts, in_parts = [], [], []
        for dx in range(_K):
            kd = _gather_cols(kwin, dx, jj1)
            prod = kd * qrow[None]
            red_parts.append(prod.reshape(_K, _NH, _HD, _W).sum(axis=2))
            sn_parts.append(_gather_cols(swin, dx, jj1))
            in_parts.append(_gather_cols(iwin, dx, jj1))
        qkdot = jnp.stack(red_parts, axis=1)
        qkdot = qkdot.transpose(2, 0, 1, 3).reshape(_NH, _L, _W)
        sn = jnp.stack(sn_parts, axis=1).reshape(_L, _S, _W)
        idn = jnp.stack(in_parts, axis=1).reshape(_L, _S, _W)

        eq = (idn[None] == irow[:, None, None, :])
        pk = jnp.where(eq, sn[None], 0.0).sum(axis=2)
        logpk = jnp.log(pk)
        pad = jnp.zeros((_S, 32 - _L, _W), jnp.float32)
        logpk_t = jnp.swapaxes(
            jnp.concatenate([logpk, pad], axis=1), 1, 2)
        qkdot_t = jnp.swapaxes(
            jnp.concatenate([qkdot, pad[:_NH]], axis=1), 1, 2)
        att_t = qkdot_t[:, None] + logpk_t[None]
        out_ref[:, :, r] = att_t[..., :_L]


def _attn_kernel(q_ref, k_ref, sims_ref, sinds_ref, out_ref):
    i = pl.program_id(0)
    nsteps = _H // _R
    args = (q_ref, k_ref, sims_ref, sinds_ref, out_ref)
    lo_lrs = [max(0, r - 2) for r in range(_R)]
    hi_base = _H - (_R + 4)
    hi_lrs = [min(_H + r - 2 - _R - hi_base, _H - _K - hi_base) for r in range(_R)]

    @pl.when(i == 0)
    def _():
        _attn_rows(*args, 0, lo_lrs, list(range(_R)))

    @pl.when(jnp.logical_and(i > 0, i < nsteps - 1))
    def _():
        _attn_rows(*args, i * _R - 2, list(range(_R)), [r + 2 for r in range(_R)])

    @pl.when(i == nsteps - 1)
    def _():
        _attn_rows(*args, hi_base, hi_lrs, [r + 4 for r in range(_R)])


def kernel(x, sims, sinds, qk_w, qk_b):
    x2 = x[0]
    sims_t = jnp.transpose(sims[0], (1, 0, 2))
    sinds_t = jnp.transpose(sinds[0], (1, 0, 2))
    b2 = qk_b.reshape(1, 2 * _C)

    nsteps = _H // _R
    q_t, k_t = pl.pallas_call(
        _proj_kernel,
        grid=(nsteps,),
        in_specs=[
            pl.BlockSpec((_R, _W, _C), lambda i: (i, 0, 0)),
            pl.BlockSpec((_C, 2 * _C), lambda i: (0, 0)),
            pl.BlockSpec((1, 2 * _C), lambda i: (0, 0)),
        ],
        out_specs=[
            pl.BlockSpec((_R, _C, _W), lambda i: (i, 0, 0)),
            pl.BlockSpec((_R, _C, _W), lambda i: (i, 0, 0)),
        ],
        out_shape=[
            jax.ShapeDtypeStruct((_H, _C, _W), jnp.float32),
            jax.ShapeDtypeStruct((_H, _C, _W), jnp.float32),
        ],
    )(x2, qk_w, b2)

    attn = pl.pallas_call(
        _attn_kernel,
        grid=(nsteps,),
        in_specs=[
            pl.BlockSpec((_R, _C, _W), lambda i: (i, 0, 0)),
            pl.BlockSpec((_H, _C, _W), lambda i: (0, 0, 0)),
            pl.BlockSpec((_H, _S, _W), lambda i: (0, 0, 0)),
            pl.BlockSpec((_H, _S, _W), lambda i: (0, 0, 0)),
        ],
        out_specs=pl.BlockSpec((_NH, _S, _R, _W, _L), lambda i: (0, 0, i, 0, 0)),
        out_shape=jax.ShapeDtypeStruct((_NH, _S, _H, _W, _L), jnp.float32),
    )(q_t, k_t, sims_t, sinds_t)

    return attn[None]
